# baseline (speedup 1.0000x reference)
import jax
import jax.numpy as jnp
from jax import lax
from jax.experimental import pallas as pl
from jax.experimental.pallas import tpu as pltpu
from jax.experimental.pallas import tpu_sc as plsc

T = 8192
D = 768
E = 64
F = 768

TM = 512
P = T + E * TM
NT = P // TM

RT = 256
RB = T // RT

NC = 2
NS = 16
NW = NC * NS
TPW = T // NW
CH = 128
NCH = TPW // CH
HD = D // 2


def _pack_rows(y):
    u = lax.bitcast_convert_type(y, jnp.uint32) + jnp.uint32(0x8000)
    lo = u[:, :HD] >> 16
    hi = u[:, HD:] & jnp.uint32(0xFFFF0000)
    return lax.bitcast_convert_type(lo | hi, jnp.int32)


def _unpack_rows(w):
    u = lax.bitcast_convert_type(w, jnp.uint32)
    lo = lax.bitcast_convert_type(u << 16, jnp.float32)
    hi = lax.bitcast_convert_type(u & jnp.uint32(0xFFFF0000), jnp.float32)
    return jnp.concatenate([lo, hi], axis=1)


def _router_body(x_ref, wg_ref, eidx_ref, gate_ref, rank_ref, counts_ref,
                 xbf_ref, cnt_scratch):
    i = pl.program_id(0)

    @pl.when(i == 0)
    def _init():
        cnt_scratch[...] = jnp.zeros((E, 1), jnp.float32)

    xb = x_ref[...]
    xbf_ref[...] = _pack_rows(xb)
    logits = lax.dot_general(wg_ref[...], xb, (((1,), (1,)), ((), ())),
                             preferred_element_type=jnp.float32)
    m = jnp.max(logits, axis=0, keepdims=True)
    s = jnp.sum(jnp.exp(logits - m), axis=0, keepdims=True)
    gate = 1.0 / s

    ii = lax.broadcasted_iota(jnp.int32, (E, RT), 0)
    eidx = jnp.min(jnp.where(logits == m, ii, E), axis=0, keepdims=True)
    oh = (ii == eidx).astype(jnp.float32)

    ri = lax.broadcasted_iota(jnp.int32, (RT, RT), 0)
    ci = lax.broadcasted_iota(jnp.int32, (RT, RT), 1)
    umat = (ri < ci).astype(jnp.float32)
    loc = jnp.dot(oh, umat, preferred_element_type=jnp.float32)

    rank = jnp.sum((loc + cnt_scratch[...]) * oh, axis=0)

    eidx_ref[0] = eidx.reshape(2, 128)
    gate_ref[0, 0, :] = gate[0]
    rank_ref[0] = rank.astype(jnp.int32).reshape(2, 128)

    cnt_scratch[...] = cnt_scratch[...] + jnp.sum(oh, axis=1, keepdims=True)
    counts_ref[...] = cnt_scratch[...]


def _router(x, wg):
    return pl.pallas_call(
        _router_body,
        grid=(RB,),
        in_specs=[
            pl.BlockSpec((RT, D), lambda i: (i, 0)),
            pl.BlockSpec((E, D), lambda i: (0, 0)),
        ],
        out_specs=[
            pl.BlockSpec((1, 2, 128), lambda i: (i, 0, 0)),
            pl.BlockSpec((1, 1, RT), lambda i: (i, 0, 0)),
            pl.BlockSpec((1, 2, 128), lambda i: (i, 0, 0)),
            pl.BlockSpec((E, 1), lambda i: (0, 0)),
            pl.BlockSpec((RT, HD), lambda i: (i, 0)),
        ],
        out_shape=[
            jax.ShapeDtypeStruct((RB, 2, 128), jnp.int32),
            jax.ShapeDtypeStruct((RB, 1, RT), jnp.float32),
            jax.ShapeDtypeStruct((RB, 2, 128), jnp.int32),
            jax.ShapeDtypeStruct((E, 1), jnp.float32),
            jax.ShapeDtypeStruct((T, HD), jnp.int32),
        ],
        scratch_shapes=[pltpu.VMEM((E, 1), jnp.float32)],
    )(x, wg)


def _dispatch_body(x_hbm, eidx_hbm, rank_hbm, ps_hbm,
                   xs_hbm, pos_hbm,
                   eidx_v, rank_v, pos_v, base_v, ps_v,
                   x_v0, x_v1,
                   sem_i, sem_x0, sem_x1, sem_sx0, sem_sx1, sem_p,
                   sem_e, sem_r):
    wid = lax.axis_index("s") * NC + lax.axis_index("c")
    base = wid * TPW

    xbuf = [x_v0, x_v1]
    sx = [sem_x0, sem_x1]
    ssx = [sem_sx0, sem_sx1]
    loads = [
        pltpu.async_copy(x_hbm.at[pl.ds(base + c * CH, CH)], xbuf[c], sx[c])
        for c in range(NCH)
    ]

    he = pltpu.async_copy(eidx_hbm.at[wid], eidx_v, sem_e)
    hr = pltpu.async_copy(rank_hbm.at[wid], rank_v, sem_r)
    pltpu.sync_copy(ps_hbm, ps_v)
    he.wait()
    gb = [pltpu.async_copy(ps_v.at[eidx_v.at[j]], base_v.at[j], sem_i)
          for j in range(2)]
    hr.wait()
    for h in gb:
        h.wait()
    for j in range(NCH):
        for k in range(CH // 16):
            sl = pl.ds(k * 16, 16)
            pos_v[j, sl] = base_v[j, sl] + rank_v[j, sl]
    hp = pltpu.async_copy(pos_v, pos_hbm.at[pl.ds(wid * NCH, NCH)], sem_p)

    scats = []
    for c in range(NCH):
        loads[c].wait()
        scats.append(pltpu.async_copy(xbuf[c], xs_hbm.at[pos_v.at[c]], ssx[c]))
    for h in scats:
        h.wait()
    hp.wait()


def _dispatch(xbf, eidx2, rank2, pad_start):
    mesh = plsc.VectorSubcoreMesh(core_axis_name="c", subcore_axis_name="s")
    f = pl.kernel(
        _dispatch_body,
        out_type=[
            jax.ShapeDtypeStruct((P, HD), jnp.int32),
            jax.ShapeDtypeStruct((T // CH, CH), jnp.int32),
        ],
        mesh=mesh,
        scratch_types=[
            pltpu.VMEM((2, 128), jnp.int32),
            pltpu.VMEM((2, 128), jnp.int32),
            pltpu.VMEM((NCH, CH), jnp.int32),
            pltpu.VMEM((2, 128), jnp.int32),
            pltpu.VMEM_SHARED((E,), jnp.int32),
            pltpu.VMEM((CH, HD), jnp.int32),
            pltpu.VMEM((CH, HD), jnp.int32),
            pltpu.SemaphoreType.DMA,
            pltpu.SemaphoreType.DMA,
            pltpu.SemaphoreType.DMA,
            pltpu.SemaphoreType.DMA,
            pltpu.SemaphoreType.DMA,
            pltpu.SemaphoreType.DMA,
            pltpu.SemaphoreType.DMA,
            pltpu.SemaphoreType.DMA,
        ],
    )
    return f(xbf, eidx2, rank2, pad_start)


def _ffn_body(gid_ref, valid_ref, xs_ref, w1_ref, b1_ref, w2_ref,
              b2_ref, ys_ref):
    i = pl.program_id(0)

    @pl.when(valid_ref[i] == 1)
    def _compute():
        xb = _unpack_rows(xs_ref[...])
        h = jnp.dot(xb, w1_ref[0], preferred_element_type=jnp.float32)
        h = jnp.maximum(h + b1_ref[0], 0.0)
        y = jnp.dot(h, w2_ref[0], preferred_element_type=jnp.float32)
        y = y + b2_ref[0]
        ys_ref[...] = _pack_rows(y)


def _ffn(tile_gid, tile_valid, xs, w1, b1r, w2, b2r):
    grid_spec = pltpu.PrefetchScalarGridSpec(
        num_scalar_prefetch=2,
        grid=(NT,),
        in_specs=[
            pl.BlockSpec((TM, HD), lambda i, g, v: (i, 0)),
            pl.BlockSpec((1, D, F), lambda i, g, v: (g[i], 0, 0)),
            pl.BlockSpec((1, 1, F), lambda i, g, v: (g[i], 0, 0)),
            pl.BlockSpec((1, F, D), lambda i, g, v: (g[i], 0, 0)),
            pl.BlockSpec((1, 1, D), lambda i, g, v: (g[i], 0, 0)),
        ],
        out_specs=pl.BlockSpec((TM, HD), lambda i, g, v: (i, 0)),
    )
    return pl.pallas_call(
        _ffn_body,
        grid_spec=grid_spec,
        out_shape=jax.ShapeDtypeStruct((P, HD), jnp.int32),
    )(tile_gid, tile_valid, xs, w1, b1r, w2, b2r)


def _combine_body(ys_hbm, pos_hbm, out_hbm, pos_v, r_v0, r_v1,
                  sem_g0, sem_g1, sem_w0, sem_w1):
    wid = lax.axis_index("s") * NC + lax.axis_index("c")
    base = wid * TPW
    pltpu.sync_copy(pos_hbm.at[pl.ds(wid * NCH, NCH)], pos_v)

    rb = [r_v0, r_v1]
    sgm = [sem_g0, sem_g1]
    swm = [sem_w0, sem_w1]

    g = [pltpu.async_copy(ys_hbm.at[pos_v.at[c]], rb[c], sgm[c])
         for c in range(NCH)]
    w = []
    for c in range(NCH):
        g[c].wait()
        w.append(pltpu.async_copy(rb[c], out_hbm.at[pl.ds(base + c * CH, CH)],
                                  swm[c]))
    for h in w:
        h.wait()


def _combine(ys, pos):
    mesh = plsc.VectorSubcoreMesh(core_axis_name="c", subcore_axis_name="s")
    f = pl.kernel(
        _combine_body,
        out_type=jax.ShapeDtypeStruct((T, HD), jnp.int32),
        mesh=mesh,
        scratch_types=[
            pltpu.VMEM((NCH, CH), jnp.int32),
            pltpu.VMEM((CH, HD), jnp.int32),
            pltpu.VMEM((CH, HD), jnp.int32),
            pltpu.SemaphoreType.DMA,
            pltpu.SemaphoreType.DMA,
            pltpu.SemaphoreType.DMA,
            pltpu.SemaphoreType.DMA,
        ],
    )
    return f(ys, pos)


def _scale_body(rows_ref, gate_ref, out_ref):
    g = gate_ref[0, 0, :].reshape(RT, 1)
    out_ref[...] = _unpack_rows(rows_ref[...]) * g


def _scale(rows, gate_col):
    return pl.pallas_call(
        _scale_body,
        grid=(RB,),
        in_specs=[
            pl.BlockSpec((RT, HD), lambda i: (i, 0)),
            pl.BlockSpec((1, 1, RT), lambda i: (i, 0, 0)),
        ],
        out_specs=pl.BlockSpec((RT, D), lambda i: (i, 0)),
        out_shape=jax.ShapeDtypeStruct((T, D), jnp.float32),
    )(rows, gate_col)


def kernel(x, Wg, W1, b1, W2, b2):
    eidx2, gate3, rank2, counts2, xbf = _router(x, Wg)
    counts = counts2[:, 0].astype(jnp.int32)

    ntiles_e = (counts + (TM - 1)) // TM
    tile_start = jnp.cumsum(ntiles_e) - ntiles_e
    pad_start = tile_start * TM
    used = jnp.sum(ntiles_e)
    tj = jnp.arange(NT, dtype=jnp.int32)
    gid = jnp.sum((tj[:, None] >= tile_start[None, :]).astype(jnp.int32),
                  axis=1) - 1
    last_e = jnp.max(jnp.where(counts > 0, jnp.arange(E, dtype=jnp.int32), 0))
    tile_gid = jnp.where(tj < used, gid, last_e).astype(jnp.int32)
    tile_valid = (tj < used).astype(jnp.int32)

    xs, pos = _dispatch(xbf, eidx2, rank2, pad_start)

    b1r = b1.reshape(E, 1, F)
    b2r = b2.reshape(E, 1, D)
    ys = _ffn(tile_gid, tile_valid, xs, W1, b1r, W2, b2r)

    rows = _combine(ys, pos)
    return _scale(rows, gate3)


# speedup vs baseline: 1.1837x; 1.1837x over previous
import jax
import jax.numpy as jnp
from jax import lax
from jax.experimental import pallas as pl
from jax.experimental.pallas import tpu as pltpu
from jax.experimental.pallas import tpu_sc as plsc

T = 8192
D = 768
E = 64
F = 768

TM = 256
P = T + E * TM
NT = P // TM

RT = 256
RB = T // RT

NC = 2
NS = 16
NW = NC * NS
TPW = T // NW
CH = 128
NCH = TPW // CH
HD = D // 2


def _pack_rows(y):
    u = lax.bitcast_convert_type(y, jnp.uint32) + jnp.uint32(0x8000)
    lo = u[:, :HD] >> 16
    hi = u[:, HD:] & jnp.uint32(0xFFFF0000)
    return lax.bitcast_convert_type(lo | hi, jnp.int32)


def _unpack_rows(w):
    u = lax.bitcast_convert_type(w, jnp.uint32)
    lo = lax.bitcast_convert_type(u << 16, jnp.float32)
    hi = lax.bitcast_convert_type(u & jnp.uint32(0xFFFF0000), jnp.float32)
    return jnp.concatenate([lo, hi], axis=1)


def _router_body(x_ref, wg_ref, eidx_ref, gate_ref, rank_ref, counts_ref,
                 xbf_ref, cnt_scratch):
    i = pl.program_id(0)

    @pl.when(i == 0)
    def _init():
        cnt_scratch[...] = jnp.zeros((E, 1), jnp.float32)

    xb = x_ref[...]
    xbf_ref[...] = _pack_rows(xb)
    logits = lax.dot_general(wg_ref[...], xb, (((1,), (1,)), ((), ())),
                             preferred_element_type=jnp.float32)
    m = jnp.max(logits, axis=0, keepdims=True)
    s = jnp.sum(jnp.exp(logits - m), axis=0, keepdims=True)
    gate = 1.0 / s

    ii = lax.broadcasted_iota(jnp.int32, (E, RT), 0)
    eidx = jnp.min(jnp.where(logits == m, ii, E), axis=0, keepdims=True)
    oh = (ii == eidx).astype(jnp.float32)

    ri = lax.broadcasted_iota(jnp.int32, (RT, RT), 0)
    ci = lax.broadcasted_iota(jnp.int32, (RT, RT), 1)
    umat = (ri < ci).astype(jnp.float32)
    loc = jnp.dot(oh, umat, preferred_element_type=jnp.float32)

    rank = jnp.sum((loc + cnt_scratch[...]) * oh, axis=0)

    eidx_ref[0] = eidx.reshape(2, 128)
    gate_ref[0, 0, :] = gate[0]
    rank_ref[0] = rank.astype(jnp.int32).reshape(2, 128)

    cnt_scratch[...] = cnt_scratch[...] + jnp.sum(oh, axis=1, keepdims=True)
    counts_ref[...] = cnt_scratch[...]


def _router(x, wg):
    return pl.pallas_call(
        _router_body,
        grid=(RB,),
        in_specs=[
            pl.BlockSpec((RT, D), lambda i: (i, 0)),
            pl.BlockSpec((E, D), lambda i: (0, 0)),
        ],
        out_specs=[
            pl.BlockSpec((1, 2, 128), lambda i: (i, 0, 0)),
            pl.BlockSpec((1, 1, RT), lambda i: (i, 0, 0)),
            pl.BlockSpec((1, 2, 128), lambda i: (i, 0, 0)),
            pl.BlockSpec((E, 1), lambda i: (0, 0)),
            pl.BlockSpec((RT, HD), lambda i: (i, 0)),
        ],
        out_shape=[
            jax.ShapeDtypeStruct((RB, 2, 128), jnp.int32),
            jax.ShapeDtypeStruct((RB, 1, RT), jnp.float32),
            jax.ShapeDtypeStruct((RB, 2, 128), jnp.int32),
            jax.ShapeDtypeStruct((E, 1), jnp.float32),
            jax.ShapeDtypeStruct((T, HD), jnp.int32),
        ],
        scratch_shapes=[pltpu.VMEM((E, 1), jnp.float32)],
    )(x, wg)


def _dispatch_body(x_hbm, eidx_hbm, rank_hbm, ps_hbm,
                   xs_hbm, pos_hbm,
                   eidx_v, rank_v, pos_v, base_v, ps_v,
                   x_v0, x_v1,
                   sem_i, sem_x0, sem_x1, sem_sx0, sem_sx1, sem_p,
                   sem_e, sem_r):
    wid = lax.axis_index("s") * NC + lax.axis_index("c")
    base = wid * TPW

    xbuf = [x_v0, x_v1]
    sx = [sem_x0, sem_x1]
    ssx = [sem_sx0, sem_sx1]
    loads = [
        pltpu.async_copy(x_hbm.at[pl.ds(base + c * CH, CH)], xbuf[c], sx[c])
        for c in range(NCH)
    ]

    he = pltpu.async_copy(eidx_hbm.at[wid], eidx_v, sem_e)
    hr = pltpu.async_copy(rank_hbm.at[wid], rank_v, sem_r)
    pltpu.sync_copy(ps_hbm, ps_v)
    he.wait()
    gb = [pltpu.async_copy(ps_v.at[eidx_v.at[j]], base_v.at[j], sem_i)
          for j in range(2)]
    hr.wait()
    for h in gb:
        h.wait()
    for j in range(NCH):
        for k in range(CH // 16):
            sl = pl.ds(k * 16, 16)
            pos_v[j, sl] = base_v[j, sl] + rank_v[j, sl]
    hp = pltpu.async_copy(pos_v, pos_hbm.at[pl.ds(wid * NCH, NCH)], sem_p)

    scats = []
    for c in range(NCH):
        loads[c].wait()
        scats.append(pltpu.async_copy(xbuf[c], xs_hbm.at[pos_v.at[c]], ssx[c]))
    for h in scats:
        h.wait()
    hp.wait()


def _dispatch(xbf, eidx2, rank2, pad_start):
    mesh = plsc.VectorSubcoreMesh(core_axis_name="c", subcore_axis_name="s")
    f = pl.kernel(
        _dispatch_body,
        out_type=[
            jax.ShapeDtypeStruct((P, HD), jnp.int32),
            jax.ShapeDtypeStruct((T // CH, CH), jnp.int32),
        ],
        mesh=mesh,
        scratch_types=[
            pltpu.VMEM((2, 128), jnp.int32),
            pltpu.VMEM((2, 128), jnp.int32),
            pltpu.VMEM((NCH, CH), jnp.int32),
            pltpu.VMEM((2, 128), jnp.int32),
            pltpu.VMEM_SHARED((E,), jnp.int32),
            pltpu.VMEM((CH, HD), jnp.int32),
            pltpu.VMEM((CH, HD), jnp.int32),
            pltpu.SemaphoreType.DMA,
            pltpu.SemaphoreType.DMA,
            pltpu.SemaphoreType.DMA,
            pltpu.SemaphoreType.DMA,
            pltpu.SemaphoreType.DMA,
            pltpu.SemaphoreType.DMA,
            pltpu.SemaphoreType.DMA,
            pltpu.SemaphoreType.DMA,
        ],
    )
    return f(xbf, eidx2, rank2, pad_start)


def _ffn_body(gid_ref, valid_ref, rid_ref, xs_ref, w1_ref, b1_ref, w2_ref,
              b2_ref, ys_ref):
    i = pl.program_id(0)

    @pl.when(valid_ref[i] == 1)
    def _compute():
        xb = _unpack_rows(xs_ref[...])
        h = jnp.dot(xb, w1_ref[0], preferred_element_type=jnp.float32)
        h = jnp.maximum(h + b1_ref[0], 0.0)
        y = jnp.dot(h, w2_ref[0], preferred_element_type=jnp.float32)
        y = y + b2_ref[0]
        ys_ref[...] = _pack_rows(y)


def _ffn(tile_gid, tile_valid, tile_rid, xs, w1, b1r, w2, b2r):
    grid_spec = pltpu.PrefetchScalarGridSpec(
        num_scalar_prefetch=3,
        grid=(NT,),
        in_specs=[
            pl.BlockSpec((TM, HD), lambda i, g, v, r: (r[i], 0)),
            pl.BlockSpec((1, D, F), lambda i, g, v, r: (g[i], 0, 0)),
            pl.BlockSpec((1, 1, F), lambda i, g, v, r: (g[i], 0, 0)),
            pl.BlockSpec((1, F, D), lambda i, g, v, r: (g[i], 0, 0)),
            pl.BlockSpec((1, 1, D), lambda i, g, v, r: (g[i], 0, 0)),
        ],
        out_specs=pl.BlockSpec((TM, HD), lambda i, g, v, r: (r[i], 0)),
    )
    return pl.pallas_call(
        _ffn_body,
        grid_spec=grid_spec,
        out_shape=jax.ShapeDtypeStruct((P, HD), jnp.int32),
    )(tile_gid, tile_valid, tile_rid, xs, w1, b1r, w2, b2r)


def _combine_body(ys_hbm, pos_hbm, out_hbm, pos_v, r_v0, r_v1,
                  sem_g0, sem_g1, sem_w0, sem_w1):
    wid = lax.axis_index("s") * NC + lax.axis_index("c")
    base = wid * TPW
    pltpu.sync_copy(pos_hbm.at[pl.ds(wid * NCH, NCH)], pos_v)

    rb = [r_v0, r_v1]
    sgm = [sem_g0, sem_g1]
    swm = [sem_w0, sem_w1]

    g = [pltpu.async_copy(ys_hbm.at[pos_v.at[c]], rb[c], sgm[c])
         for c in range(NCH)]
    w = []
    for c in range(NCH):
        g[c].wait()
        w.append(pltpu.async_copy(rb[c], out_hbm.at[pl.ds(base + c * CH, CH)],
                                  swm[c]))
    for h in w:
        h.wait()


def _combine(ys, pos):
    mesh = plsc.VectorSubcoreMesh(core_axis_name="c", subcore_axis_name="s")
    f = pl.kernel(
        _combine_body,
        out_type=jax.ShapeDtypeStruct((T, HD), jnp.int32),
        mesh=mesh,
        scratch_types=[
            pltpu.VMEM((NCH, CH), jnp.int32),
            pltpu.VMEM((CH, HD), jnp.int32),
            pltpu.VMEM((CH, HD), jnp.int32),
            pltpu.SemaphoreType.DMA,
            pltpu.SemaphoreType.DMA,
            pltpu.SemaphoreType.DMA,
            pltpu.SemaphoreType.DMA,
        ],
    )
    return f(ys, pos)


def _scale_body(rows_ref, gate_ref, out_ref):
    g = gate_ref[0, 0, :].reshape(RT, 1)
    out_ref[...] = _unpack_rows(rows_ref[...]) * g


def _scale(rows, gate_col):
    return pl.pallas_call(
        _scale_body,
        grid=(RB,),
        in_specs=[
            pl.BlockSpec((RT, HD), lambda i: (i, 0)),
            pl.BlockSpec((1, 1, RT), lambda i: (i, 0, 0)),
        ],
        out_specs=pl.BlockSpec((RT, D), lambda i: (i, 0)),
        out_shape=jax.ShapeDtypeStruct((T, D), jnp.float32),
    )(rows, gate_col)


def kernel(x, Wg, W1, b1, W2, b2):
    eidx2, gate3, rank2, counts2, xbf = _router(x, Wg)
    counts = counts2[:, 0].astype(jnp.int32)

    ntiles_e = (counts + (TM - 1)) // TM
    tile_start = jnp.cumsum(ntiles_e) - ntiles_e
    pad_start = tile_start * TM
    used = jnp.sum(ntiles_e)
    tj = jnp.arange(NT, dtype=jnp.int32)
    gid = jnp.sum((tj[:, None] >= tile_start[None, :]).astype(jnp.int32),
                  axis=1) - 1
    last_e = jnp.max(jnp.where(counts > 0, jnp.arange(E, dtype=jnp.int32), 0))
    tile_gid = jnp.where(tj < used, gid, last_e).astype(jnp.int32)
    tile_valid = (tj < used).astype(jnp.int32)
    tile_rid = jnp.minimum(tj, used - 1).astype(jnp.int32)

    xs, pos = _dispatch(xbf, eidx2, rank2, pad_start)

    b1r = b1.reshape(E, 1, F)
    b2r = b2.reshape(E, 1, D)
    ys = _ffn(tile_gid, tile_valid, tile_rid, xs, W1, b1r, W2, b2r)

    rows = _combine(ys, pos)
    return _scale(rows, gate3)


# speedup vs baseline: 1.2824x; 1.0834x over previous
import jax
import jax.numpy as jnp
from jax import lax
from jax.experimental import pallas as pl
from jax.experimental.pallas import tpu as pltpu
from jax.experimental.pallas import tpu_sc as plsc

T = 8192
D = 768
E = 64
F = 768

TM = 256
P = T + E * TM
NT = P // TM

RT = 512
RB = T // RT

NC = 2
NS = 16
NW = NC * NS
TPW = T // NW
CH = 128
NCH = TPW // CH
HD = D // 2


def _pack_rows(y):
    u = lax.bitcast_convert_type(y, jnp.uint32) + jnp.uint32(0x8000)
    lo = u[:, :HD] >> 16
    hi = u[:, HD:] & jnp.uint32(0xFFFF0000)
    return lax.bitcast_convert_type(lo | hi, jnp.int32)


def _unpack_rows(w):
    u = lax.bitcast_convert_type(w, jnp.uint32)
    lo = lax.bitcast_convert_type(u << 16, jnp.float32)
    hi = lax.bitcast_convert_type(u & jnp.uint32(0xFFFF0000), jnp.float32)
    return jnp.concatenate([lo, hi], axis=1)


def _router_body(x_ref, wg_ref, eidx_ref, gate_ref, rank_ref, counts_ref,
                 xbf_ref, cnt_scratch):
    i = pl.program_id(0)

    @pl.when(i == 0)
    def _init():
        cnt_scratch[...] = jnp.zeros((E, 1), jnp.float32)

    xb = x_ref[...]
    xbf_ref[...] = _pack_rows(xb)
    logits = lax.dot_general(wg_ref[...], xb, (((1,), (1,)), ((), ())),
                             preferred_element_type=jnp.float32)
    m = jnp.max(logits, axis=0, keepdims=True)
    s = jnp.sum(jnp.exp(logits - m), axis=0, keepdims=True)
    gate = 1.0 / s

    ii = lax.broadcasted_iota(jnp.int32, (E, RT), 0)
    eidx = jnp.min(jnp.where(logits == m, ii, E), axis=0, keepdims=True)
    oh = (ii == eidx).astype(jnp.float32)

    ri = lax.broadcasted_iota(jnp.int32, (RT, RT), 0)
    ci = lax.broadcasted_iota(jnp.int32, (RT, RT), 1)
    umat = (ri < ci).astype(jnp.float32)
    loc = jnp.dot(oh, umat, preferred_element_type=jnp.float32)

    rank = jnp.sum((loc + cnt_scratch[...]) * oh, axis=0)

    eidx_ref[0] = eidx.reshape(RT // 128, 128)
    gate_ref[0, 0, :] = gate[0]
    rank_ref[0] = rank.astype(jnp.int32).reshape(RT // 128, 128)

    cnt_scratch[...] = cnt_scratch[...] + jnp.sum(oh, axis=1, keepdims=True)
    counts_ref[...] = cnt_scratch[...]


def _router(x, wg):
    return pl.pallas_call(
        _router_body,
        grid=(RB,),
        in_specs=[
            pl.BlockSpec((RT, D), lambda i: (i, 0)),
            pl.BlockSpec((E, D), lambda i: (0, 0)),
        ],
        out_specs=[
            pl.BlockSpec((1, RT // 128, 128), lambda i: (i, 0, 0)),
            pl.BlockSpec((1, 1, RT), lambda i: (i, 0, 0)),
            pl.BlockSpec((1, RT // 128, 128), lambda i: (i, 0, 0)),
            pl.BlockSpec((E, 1), lambda i: (0, 0)),
            pl.BlockSpec((RT, HD), lambda i: (i, 0)),
        ],
        out_shape=[
            jax.ShapeDtypeStruct((RB, RT // 128, 128), jnp.int32),
            jax.ShapeDtypeStruct((RB, 1, RT), jnp.float32),
            jax.ShapeDtypeStruct((RB, RT // 128, 128), jnp.int32),
            jax.ShapeDtypeStruct((E, 1), jnp.float32),
            jax.ShapeDtypeStruct((T, HD), jnp.int32),
        ],
        scratch_shapes=[pltpu.VMEM((E, 1), jnp.float32)],
    )(x, wg)


def _dispatch_body(x_hbm, eidx_hbm, rank_hbm, ps_hbm,
                   xs_hbm, pos_hbm,
                   eidx_v, rank_v, pos_v, base_v, ps_v,
                   x_v0, x_v1,
                   sem_i, sem_x0, sem_x1, sem_sx0, sem_sx1, sem_p,
                   sem_e, sem_r):
    wid = lax.axis_index("s") * NC + lax.axis_index("c")
    base = wid * TPW

    xbuf = [x_v0, x_v1]
    sx = [sem_x0, sem_x1]
    ssx = [sem_sx0, sem_sx1]
    loads = [
        pltpu.async_copy(x_hbm.at[pl.ds(base + c * CH, CH)], xbuf[c], sx[c])
        for c in range(NCH)
    ]

    he = pltpu.async_copy(
        eidx_hbm.at[wid // 2, pl.ds((wid % 2) * 2, 2)], eidx_v, sem_e)
    hr = pltpu.async_copy(
        rank_hbm.at[wid // 2, pl.ds((wid % 2) * 2, 2)], rank_v, sem_r)
    pltpu.sync_copy(ps_hbm, ps_v)
    he.wait()
    gb = [pltpu.async_copy(ps_v.at[eidx_v.at[j]], base_v.at[j], sem_i)
          for j in range(2)]
    hr.wait()
    for h in gb:
        h.wait()
    for j in range(NCH):
        for k in range(CH // 16):
            sl = pl.ds(k * 16, 16)
            pos_v[j, sl] = base_v[j, sl] + rank_v[j, sl]
    hp = pltpu.async_copy(pos_v, pos_hbm.at[pl.ds(wid * NCH, NCH)], sem_p)

    scats = []
    for c in range(NCH):
        loads[c].wait()
        scats.append(pltpu.async_copy(xbuf[c], xs_hbm.at[pos_v.at[c]], ssx[c]))
    for h in scats:
        h.wait()
    hp.wait()


def _dispatch(xbf, eidx2, rank2, pad_start):
    mesh = plsc.VectorSubcoreMesh(core_axis_name="c", subcore_axis_name="s")
    f = pl.kernel(
        _dispatch_body,
        out_type=[
            jax.ShapeDtypeStruct((P, HD), jnp.int32),
            jax.ShapeDtypeStruct((T // CH, CH), jnp.int32),
        ],
        mesh=mesh,
        scratch_types=[
            pltpu.VMEM((2, 128), jnp.int32),
            pltpu.VMEM((2, 128), jnp.int32),
            pltpu.VMEM((NCH, CH), jnp.int32),
            pltpu.VMEM((2, 128), jnp.int32),
            pltpu.VMEM_SHARED((E,), jnp.int32),
            pltpu.VMEM((CH, HD), jnp.int32),
            pltpu.VMEM((CH, HD), jnp.int32),
            pltpu.SemaphoreType.DMA,
            pltpu.SemaphoreType.DMA,
            pltpu.SemaphoreType.DMA,
            pltpu.SemaphoreType.DMA,
            pltpu.SemaphoreType.DMA,
            pltpu.SemaphoreType.DMA,
            pltpu.SemaphoreType.DMA,
            pltpu.SemaphoreType.DMA,
        ],
    )
    return f(xbf, eidx2, rank2, pad_start)


def _ffn_body(gid_ref, valid_ref, rid_ref, xs_ref, w1_ref, b1_ref, w2_ref,
              b2_ref, ys_ref):
    i = pl.program_id(0)

    @pl.when(valid_ref[i] == 1)
    def _compute():
        xb = _unpack_rows(xs_ref[...])
        h = jnp.dot(xb, w1_ref[0], preferred_element_type=jnp.float32)
        h = jnp.maximum(h + b1_ref[0], 0.0)
        y = jnp.dot(h, w2_ref[0], preferred_element_type=jnp.float32)
        y = y + b2_ref[0]
        ys_ref[...] = _pack_rows(y)


def _ffn(tile_gid, tile_valid, tile_rid, xs, w1, b1r, w2, b2r):
    grid_spec = pltpu.PrefetchScalarGridSpec(
        num_scalar_prefetch=3,
        grid=(NT,),
        in_specs=[
            pl.BlockSpec((TM, HD), lambda i, g, v, r: (r[i], 0)),
            pl.BlockSpec((1, D, F), lambda i, g, v, r: (g[i], 0, 0)),
            pl.BlockSpec((1, 1, F), lambda i, g, v, r: (g[i], 0, 0)),
            pl.BlockSpec((1, F, D), lambda i, g, v, r: (g[i], 0, 0)),
            pl.BlockSpec((1, 1, D), lambda i, g, v, r: (g[i], 0, 0)),
        ],
        out_specs=pl.BlockSpec((TM, HD), lambda i, g, v, r: (r[i], 0)),
    )
    return pl.pallas_call(
        _ffn_body,
        grid_spec=grid_spec,
        out_shape=jax.ShapeDtypeStruct((P, HD), jnp.int32),
    )(tile_gid, tile_valid, tile_rid, xs, w1, b1r, w2, b2r)


def _combine_body(ys_hbm, pos_hbm, out_hbm, pos_v, r_v0, r_v1,
                  sem_g0, sem_g1, sem_w0, sem_w1):
    wid = lax.axis_index("s") * NC + lax.axis_index("c")
    base = wid * TPW
    pltpu.sync_copy(pos_hbm.at[pl.ds(wid * NCH, NCH)], pos_v)

    rb = [r_v0, r_v1]
    sgm = [sem_g0, sem_g1]
    swm = [sem_w0, sem_w1]

    g = [pltpu.async_copy(ys_hbm.at[pos_v.at[c]], rb[c], sgm[c])
         for c in range(NCH)]
    w = []
    for c in range(NCH):
        g[c].wait()
        w.append(pltpu.async_copy(rb[c], out_hbm.at[pl.ds(base + c * CH, CH)],
                                  swm[c]))
    for h in w:
        h.wait()


def _combine(ys, pos):
    mesh = plsc.VectorSubcoreMesh(core_axis_name="c", subcore_axis_name="s")
    f = pl.kernel(
        _combine_body,
        out_type=jax.ShapeDtypeStruct((T, HD), jnp.int32),
        mesh=mesh,
        scratch_types=[
            pltpu.VMEM((NCH, CH), jnp.int32),
            pltpu.VMEM((CH, HD), jnp.int32),
            pltpu.VMEM((CH, HD), jnp.int32),
            pltpu.SemaphoreType.DMA,
            pltpu.SemaphoreType.DMA,
            pltpu.SemaphoreType.DMA,
            pltpu.SemaphoreType.DMA,
        ],
    )
    return f(ys, pos)


def _scale_body(rows_ref, gate_ref, out_ref):
    g = gate_ref[0, 0, :].reshape(RT, 1)
    out_ref[...] = _unpack_rows(rows_ref[...]) * g


def _scale(rows, gate_col):
    return pl.pallas_call(
        _scale_body,
        grid=(RB,),
        in_specs=[
            pl.BlockSpec((RT, HD), lambda i: (i, 0)),
            pl.BlockSpec((1, 1, RT), lambda i: (i, 0, 0)),
        ],
        out_specs=pl.BlockSpec((RT, D), lambda i: (i, 0)),
        out_shape=jax.ShapeDtypeStruct((T, D), jnp.float32),
    )(rows, gate_col)


def kernel(x, Wg, W1, b1, W2, b2):
    eidx2, gate3, rank2, counts2, xbf = _router(x, Wg)
    counts = counts2[:, 0].astype(jnp.int32)

    ntiles_e = (counts + (TM - 1)) // TM
    tile_start = jnp.cumsum(ntiles_e) - ntiles_e
    pad_start = tile_start * TM
    used = jnp.sum(ntiles_e)
    tj = jnp.arange(NT, dtype=jnp.int32)
    gid = jnp.sum((tj[:, None] >= tile_start[None, :]).astype(jnp.int32),
                  axis=1) - 1
    last_e = jnp.max(jnp.where(counts > 0, jnp.arange(E, dtype=jnp.int32), 0))
    tile_gid = jnp.where(tj < used, gid, last_e).astype(jnp.int32)
    tile_valid = (tj < used).astype(jnp.int32)
    tile_rid = jnp.minimum(tj, used - 1).astype(jnp.int32)

    xs, pos = _dispatch(xbf, eidx2, rank2, pad_start)

    b1r = b1.reshape(E, 1, F)
    b2r = b2.reshape(E, 1, D)
    ys = _ffn(tile_gid, tile_valid, tile_rid, xs, W1, b1r, W2, b2r)

    rows = _combine(ys, pos)
    return _scale(rows, gate3)
